# baseline (speedup 1.0000x reference)
import functools

import jax
import jax.numpy as jnp
from jax import lax
from jax.experimental import pallas as pl
from jax.experimental.pallas import tpu as pltpu
from jax.experimental.pallas import tpu_sc as plsc

_N = 10000
_E = 320000
_D = 128
_NC = 2
_NS = 16
_NW = _NC * _NS
_EPW = _E // _NW
_BLK = 128
_NBLK = _E // _BLK
_BASE_BLKS = _NBLK // _NW
_EXTRA_WORKERS = _NBLK % _NW
_NPAD = 10240
_RPS = _NPAD // _NS

_mesh = plsc.VectorSubcoreMesh(
    core_axis_name="c", subcore_axis_name="s", num_cores=_NC, num_subcores=_NS
)


def _deg_body(dst_hbm, out_hbm, idx_v, loc_v):
    c = lax.axis_index("c")
    s = lax.axis_index("s")
    wid = c * _NS + s

    def zero_body(i, carry):
        loc_v[pl.ds(i * 16, 16)] = jnp.zeros((16,), jnp.float32)
        return carry

    lax.fori_loop(0, _N // 16, zero_body, 0)

    pltpu.sync_copy(dst_hbm.at[pl.ds(wid * _EPW, _EPW)], idx_v)
    ones = jnp.ones((16,), jnp.float32)

    def hist_body(i, carry):
        idx = idx_v[pl.ds(i * 16, 16)]
        plsc.addupdate_scatter(loc_v, [idx], ones)
        return carry

    lax.fori_loop(0, _EPW // 16, hist_body, 0)
    pltpu.sync_copy(loc_v, out_hbm.at[wid])


_deg_call = pl.kernel(
    _deg_body,
    out_type=jax.ShapeDtypeStruct((_NW, _N), jnp.float32),
    mesh=_mesh,
    compiler_params=pltpu.CompilerParams(needs_layout_passes=False),
    scratch_types=[
        pltpu.VMEM((_EPW,), jnp.int32),
        pltpu.VMEM((_N,), jnp.float32),
    ],
)


_EBLK = 80
_BPW = 128
_EPAD = _BPW * _NW * _EBLK
_NBUF = 2


def _scatter_body(hp_hbm, src_hbm, dst_hbm, zeros_hbm, out_hbm,
                  srcix_v, dstix_v, r0, r1, g0, g1, s0, s1, acc_sh):
    rows = (r0, r1)
    gsems = (g0, g1)
    ssems = (s0, s1)
    c = lax.axis_index("c")
    s = lax.axis_index("s")
    wid = c * _NS + s

    pltpu.sync_copy(src_hbm.at[pl.ds(wid * _BPW, _BPW)], srcix_v)
    pltpu.sync_copy(dst_hbm.at[pl.ds(wid * _BPW, _BPW)], dstix_v)

    pltpu.sync_copy(zeros_hbm.at[pl.ds(s * _RPS, _RPS)],
                    acc_sh.at[pl.ds(s * _RPS, _RPS)])
    plsc.subcore_barrier()

    for b in range(_NBUF):
        pltpu.async_copy(hp_hbm.at[srcix_v.at[b]], rows[b], gsems[b])

    nsteps = _BPW // _NBUF

    def step(si, carry):
        for b in range(_NBUF):
            j = si * _NBUF + b
            pltpu.make_async_copy(hp_hbm.at[srcix_v.at[j]],
                                  rows[b], gsems[b]).wait()
            sc = pltpu.async_copy(rows[b], acc_sh.at[dstix_v.at[j]],
                                  ssems[b], add=True)

            @pl.when(si < nsteps - 1)
            def _():
                sc.wait()
                pltpu.async_copy(hp_hbm.at[srcix_v.at[j + _NBUF]],
                                 rows[b], gsems[b])

        return carry

    lax.fori_loop(0, nsteps, step, 0)
    for b in range(_NBUF):
        pltpu.make_async_copy(rows[b], acc_sh.at[dstix_v.at[0]],
                              ssems[b]).wait()
    plsc.subcore_barrier()
    pltpu.sync_copy(acc_sh.at[pl.ds(s * _RPS, _RPS)],
                    out_hbm.at[c, pl.ds(s * _RPS, _RPS)])


_scatter_call = pl.kernel(
    _scatter_body,
    out_type=jax.ShapeDtypeStruct((_NC, _NPAD, _D), jnp.float32),
    mesh=_mesh,
    compiler_params=pltpu.CompilerParams(
        needs_layout_passes=False, use_tc_tiling_on_sc=False),
    scratch_types=[
        pltpu.VMEM((_BPW, _EBLK), jnp.int32),
        pltpu.VMEM((_BPW, _EBLK), jnp.int32),
        pltpu.VMEM((_EBLK, _D), jnp.float32),
        pltpu.VMEM((_EBLK, _D), jnp.float32),
        pltpu.SemaphoreType.DMA,
        pltpu.SemaphoreType.DMA,
        pltpu.SemaphoreType.DMA,
        pltpu.SemaphoreType.DMA,
        pltpu.VMEM_SHARED((_NPAD, _D), jnp.float32),
    ],
)


_RB = 1000
_GRID = _N // _RB


def _tc_first_body(x_ref, deg_ref, gb_ref, w_ref, hp_ref, dinv_ref):
    deg = jnp.sum(deg_ref[...], axis=0) + 1.0
    dinv = lax.rsqrt(deg)
    h = x_ref[...] * gb_ref[0:1, :] + gb_ref[1:2, :]
    hp = jnp.dot(h, w_ref[...], preferred_element_type=jnp.float32)
    hp_ref[...] = hp * dinv
    dinv_ref[...] = dinv


def _tc_first(x, deg_p, gb, w1):
    return pl.pallas_call(
        _tc_first_body,
        grid=(_GRID,),
        in_specs=[
            pl.BlockSpec((_RB, _D), lambda i: (i, 0)),
            pl.BlockSpec((_NW, _RB, 1), lambda i: (0, i, 0)),
            pl.BlockSpec((2, _D), lambda i: (0, 0)),
            pl.BlockSpec((_D, _D), lambda i: (0, 0)),
        ],
        out_specs=[
            pl.BlockSpec((_RB, _D), lambda i: (i, 0)),
            pl.BlockSpec((_RB, 1), lambda i: (i, 0)),
        ],
        out_shape=[
            jax.ShapeDtypeStruct((_N, _D), jnp.float32),
            jax.ShapeDtypeStruct((_N, 1), jnp.float32),
        ],
    )(x, deg_p, gb, w1)


def _tc_mid_body(acc_ref, hp_ref, dinv_ref, b_ref, w_ref, out_ref):
    dinv = dinv_ref[...]
    z = (acc_ref[0] + acc_ref[1] + hp_ref[...]) * dinv + b_ref[...]
    h = jnp.maximum(z, 0.0)
    out_ref[...] = jnp.dot(h, w_ref[...], preferred_element_type=jnp.float32) * dinv


def _tc_mid(acc_p, hp, dinv, b, w_next):
    return pl.pallas_call(
        _tc_mid_body,
        grid=(_GRID,),
        in_specs=[
            pl.BlockSpec((_NC, _RB, _D), lambda i: (0, i, 0)),
            pl.BlockSpec((_RB, _D), lambda i: (i, 0)),
            pl.BlockSpec((_RB, 1), lambda i: (i, 0)),
            pl.BlockSpec((1, _D), lambda i: (0, 0)),
            pl.BlockSpec((_D, _D), lambda i: (0, 0)),
        ],
        out_specs=pl.BlockSpec((_RB, _D), lambda i: (i, 0)),
        out_shape=jax.ShapeDtypeStruct((_N, _D), jnp.float32),
    )(acc_p, hp, dinv, b, w_next)


def _tc_final_body(acc_ref, hp_ref, dinv_ref, b_ref, wl_ref, bl_ref,
                   out_ref, sum_ref):
    i = pl.program_id(0)
    z = (acc_ref[0] + acc_ref[1] + hp_ref[...]) * dinv_ref[...] + b_ref[...]
    h = jnp.maximum(z, 0.0)
    part = jnp.sum(h, axis=0, keepdims=True)

    @pl.when(i == 0)
    def _():
        sum_ref[...] = part

    @pl.when(i > 0)
    def _():
        sum_ref[...] += part

    @pl.when(i == _GRID - 1)
    def _():
        m = sum_ref[...] * (1.0 / _N)
        logits = jnp.dot(m, wl_ref[...], preferred_element_type=jnp.float32)
        logits = logits + bl_ref[...]
        zmax = jnp.max(logits, axis=1, keepdims=True)
        e = jnp.exp(logits - zmax)
        out_ref[...] = e / jnp.sum(e, axis=1, keepdims=True)


def _tc_final(acc_p, hp, dinv, b, wl, bl):
    return pl.pallas_call(
        _tc_final_body,
        grid=(_GRID,),
        in_specs=[
            pl.BlockSpec((_NC, _RB, _D), lambda i: (0, i, 0)),
            pl.BlockSpec((_RB, _D), lambda i: (i, 0)),
            pl.BlockSpec((_RB, 1), lambda i: (i, 0)),
            pl.BlockSpec((1, _D), lambda i: (0, 0)),
            pl.BlockSpec((_D, 2), lambda i: (0, 0)),
            pl.BlockSpec((1, 2), lambda i: (0, 0)),
        ],
        out_specs=pl.BlockSpec((1, 2), lambda i: (0, 0)),
        out_shape=jax.ShapeDtypeStruct((1, 2), jnp.float32),
        scratch_shapes=[pltpu.VMEM((1, _D), jnp.float32)],
    )(acc_p, hp, dinv, b, wl, bl)


def kernel(x, edge_index, bn_gamma, bn_beta, W1, b1, W2, b2, W3, b3, Wl, bl):
    ei = edge_index.astype(jnp.int32)
    src = ei[0]
    dst = ei[1]

    eps = 1e-5
    gb = jnp.stack([bn_gamma * (1.0 / jnp.sqrt(1.0 + eps)), bn_beta])

    deg_p = _deg_call(dst).reshape(_NW, _N, 1)
    zeros = jnp.zeros((_NPAD, _D), jnp.float32)

    npad_e = _EPAD - _E
    src_p = jnp.concatenate(
        [src, jnp.zeros((npad_e,), jnp.int32)]).reshape(_NW * _BPW, _EBLK)
    dst_p = jnp.concatenate(
        [dst, jnp.full((npad_e,), _N, jnp.int32)]).reshape(_NW * _BPW, _EBLK)

    hp1, dinv = _tc_first(x, deg_p, gb, W1)
    acc1 = _scatter_call(hp1, src_p, dst_p, zeros)
    hp2 = _tc_mid(acc1, hp1, dinv, b1.reshape(1, _D), W2)
    acc2 = _scatter_call(hp2, src_p, dst_p, zeros)
    hp3 = _tc_mid(acc2, hp2, dinv, b2.reshape(1, _D), W3)
    acc3 = _scatter_call(hp3, src_p, dst_p, zeros)
    return _tc_final(acc3, hp3, dinv, b3.reshape(1, _D), Wl,
                     bl.reshape(1, 2))


# speedup vs baseline: 2.8314x; 2.8314x over previous
import functools

import jax
import jax.numpy as jnp
from jax import lax
from jax.experimental import pallas as pl
from jax.experimental.pallas import tpu as pltpu
from jax.experimental.pallas import tpu_sc as plsc

_N = 10000
_E = 320000
_D = 128
_NC = 2
_NS = 16
_NW = _NC * _NS
_EPW = _E // _NW
_BLK = 128
_NBLK = _E // _BLK
_BASE_BLKS = _NBLK // _NW
_EXTRA_WORKERS = _NBLK % _NW
_NPAD = 10240
_RPS = _NPAD // _NS

_mesh = plsc.VectorSubcoreMesh(
    core_axis_name="c", subcore_axis_name="s", num_cores=_NC, num_subcores=_NS
)


def _deg_body(dst_hbm, out_hbm, idx_v, loc_v):
    c = lax.axis_index("c")
    s = lax.axis_index("s")
    wid = c * _NS + s

    def zero_body(i, carry):
        loc_v[pl.ds(i * 16, 16)] = jnp.zeros((16,), jnp.float32)
        return carry

    lax.fori_loop(0, _N // 16, zero_body, 0)

    pltpu.sync_copy(dst_hbm.at[pl.ds(wid * _EPW, _EPW)], idx_v)
    ones = jnp.ones((16,), jnp.float32)

    def hist_body(i, carry):
        idx = idx_v[pl.ds(i * 16, 16)]
        plsc.addupdate_scatter(loc_v, [idx], ones)
        return carry

    lax.fori_loop(0, _EPW // 16, hist_body, 0)
    pltpu.sync_copy(loc_v, out_hbm.at[wid])


_deg_call = pl.kernel(
    _deg_body,
    out_type=jax.ShapeDtypeStruct((_NW, _N), jnp.float32),
    mesh=_mesh,
    compiler_params=pltpu.CompilerParams(needs_layout_passes=False),
    scratch_types=[
        pltpu.VMEM((_EPW,), jnp.int32),
        pltpu.VMEM((_N,), jnp.float32),
    ],
)


_EBLK = 80
_BPW = 128
_EPAD = _BPW * _NW * _EBLK
_NBUF = 2


def _scatter_body(hp_hbm, src_hbm, dst_hbm, zeros_hbm, out_hbm,
                  srcix_v, dstix_v, r0, r1, g0, g1, s0, s1, acc_sh):
    rows = (r0, r1)
    gsems = (g0, g1)
    ssems = (s0, s1)
    c = lax.axis_index("c")
    s = lax.axis_index("s")
    wid = c * _NS + s

    pltpu.sync_copy(src_hbm.at[pl.ds(wid * _BPW, _BPW)], srcix_v)
    pltpu.sync_copy(dst_hbm.at[pl.ds(wid * _BPW, _BPW)], dstix_v)

    pltpu.sync_copy(zeros_hbm.at[pl.ds(s * _RPS, _RPS)],
                    acc_sh.at[pl.ds(s * _RPS, _RPS)])
    plsc.subcore_barrier()

    for b in range(_NBUF):
        pltpu.async_copy(hp_hbm.at[srcix_v.at[b]], rows[b], gsems[b])

    nsteps = _BPW // _NBUF

    def step(si, carry):
        for b in range(_NBUF):
            j = si * _NBUF + b
            pltpu.make_async_copy(hp_hbm.at[srcix_v.at[j]],
                                  rows[b], gsems[b]).wait()
            sc = pltpu.async_copy(rows[b], acc_sh.at[dstix_v.at[j]],
                                  ssems[b], add=True)

            @pl.when(si < nsteps - 1)
            def _():
                sc.wait()
                pltpu.async_copy(hp_hbm.at[srcix_v.at[j + _NBUF]],
                                 rows[b], gsems[b])

        return carry

    lax.fori_loop(0, nsteps, step, 0)
    for b in range(_NBUF):
        pltpu.make_async_copy(rows[b], acc_sh.at[dstix_v.at[0]],
                              ssems[b]).wait()
    plsc.subcore_barrier()
    pltpu.sync_copy(acc_sh.at[pl.ds(s * _RPS, _RPS)],
                    out_hbm.at[c, pl.ds(s * _RPS, _RPS)])


_scatter_call = pl.kernel(
    _scatter_body,
    out_type=jax.ShapeDtypeStruct((_NC, _NPAD, _D), jnp.float32),
    mesh=_mesh,
    compiler_params=pltpu.CompilerParams(
        needs_layout_passes=False, use_tc_tiling_on_sc=False),
    scratch_types=[
        pltpu.VMEM((_BPW, _EBLK), jnp.int32),
        pltpu.VMEM((_BPW, _EBLK), jnp.int32),
        pltpu.VMEM((_EBLK, _D), jnp.float32),
        pltpu.VMEM((_EBLK, _D), jnp.float32),
        pltpu.SemaphoreType.DMA,
        pltpu.SemaphoreType.DMA,
        pltpu.SemaphoreType.DMA,
        pltpu.SemaphoreType.DMA,
        pltpu.VMEM_SHARED((_NPAD, _D), jnp.float32),
    ],
)


_RB = 1000
_GRID = _N // _RB


def _tc_first_body(x_ref, deg_ref, gb_ref, w_ref, hp_ref, dinv_ref):
    deg = jnp.sum(deg_ref[...], axis=0) + 1.0
    dinv = lax.rsqrt(deg)
    h = x_ref[...] * gb_ref[0:1, :] + gb_ref[1:2, :]
    hp = jnp.dot(h, w_ref[...], preferred_element_type=jnp.float32)
    hp_ref[...] = hp * dinv
    dinv_ref[...] = dinv


def _tc_first(x, deg_p, gb, w1):
    return pl.pallas_call(
        _tc_first_body,
        grid=(_GRID,),
        in_specs=[
            pl.BlockSpec((_RB, _D), lambda i: (i, 0)),
            pl.BlockSpec((_NW, _RB, 1), lambda i: (0, i, 0)),
            pl.BlockSpec((2, _D), lambda i: (0, 0)),
            pl.BlockSpec((_D, _D), lambda i: (0, 0)),
        ],
        out_specs=[
            pl.BlockSpec((_RB, _D), lambda i: (i, 0)),
            pl.BlockSpec((_RB, 1), lambda i: (i, 0)),
        ],
        out_shape=[
            jax.ShapeDtypeStruct((_N, _D), jnp.float32),
            jax.ShapeDtypeStruct((_N, 1), jnp.float32),
        ],
    )(x, deg_p, gb, w1)


def _tc_mid_body(acc_ref, hp_ref, dinv_ref, b_ref, w_ref, out_ref):
    dinv = dinv_ref[...]
    z = (acc_ref[0] + acc_ref[1] + hp_ref[...]) * dinv + b_ref[...]
    h = jnp.maximum(z, 0.0)
    out_ref[...] = jnp.dot(h, w_ref[...], preferred_element_type=jnp.float32) * dinv


def _tc_mid(acc_p, hp, dinv, b, w_next):
    return pl.pallas_call(
        _tc_mid_body,
        grid=(_GRID,),
        in_specs=[
            pl.BlockSpec((_NC, _RB, _D), lambda i: (0, i, 0)),
            pl.BlockSpec((_RB, _D), lambda i: (i, 0)),
            pl.BlockSpec((_RB, 1), lambda i: (i, 0)),
            pl.BlockSpec((1, _D), lambda i: (0, 0)),
            pl.BlockSpec((_D, _D), lambda i: (0, 0)),
        ],
        out_specs=pl.BlockSpec((_RB, _D), lambda i: (i, 0)),
        out_shape=jax.ShapeDtypeStruct((_N, _D), jnp.float32),
    )(acc_p, hp, dinv, b, w_next)


def _tc_final_body(acc_ref, hp_ref, dinv_ref, b_ref, wl_ref, bl_ref,
                   out_ref, sum_ref):
    i = pl.program_id(0)
    z = (acc_ref[0] + acc_ref[1] + hp_ref[...]) * dinv_ref[...] + b_ref[...]
    h = jnp.maximum(z, 0.0)
    part = jnp.sum(h, axis=0, keepdims=True)

    @pl.when(i == 0)
    def _():
        sum_ref[...] = part

    @pl.when(i > 0)
    def _():
        sum_ref[...] += part

    @pl.when(i == _GRID - 1)
    def _():
        m = sum_ref[...] * (1.0 / _N)
        logits = jnp.dot(m, wl_ref[...], preferred_element_type=jnp.float32)
        logits = logits + bl_ref[...]
        zmax = jnp.max(logits, axis=1, keepdims=True)
        e = jnp.exp(logits - zmax)
        out_ref[...] = e / jnp.sum(e, axis=1, keepdims=True)


def _tc_final(acc_p, hp, dinv, b, wl, bl):
    return pl.pallas_call(
        _tc_final_body,
        grid=(_GRID,),
        in_specs=[
            pl.BlockSpec((_NC, _RB, _D), lambda i: (0, i, 0)),
            pl.BlockSpec((_RB, _D), lambda i: (i, 0)),
            pl.BlockSpec((_RB, 1), lambda i: (i, 0)),
            pl.BlockSpec((1, _D), lambda i: (0, 0)),
            pl.BlockSpec((_D, 2), lambda i: (0, 0)),
            pl.BlockSpec((1, 2), lambda i: (0, 0)),
        ],
        out_specs=pl.BlockSpec((1, 2), lambda i: (0, 0)),
        out_shape=jax.ShapeDtypeStruct((1, 2), jnp.float32),
        scratch_shapes=[pltpu.VMEM((1, _D), jnp.float32)],
    )(acc_p, hp, dinv, b, wl, bl)


def kernel(x, edge_index, bn_gamma, bn_beta, W1, b1, W2, b2, W3, b3, Wl, bl):
    ei = edge_index.astype(jnp.int32)
    src = ei[0]
    dst = ei[1]

    eps = 1e-5
    gb = jnp.stack([bn_gamma * (1.0 / jnp.sqrt(1.0 + eps)), bn_beta])

    deg_p = _deg_call(dst).reshape(_NW, _N, 1)
    zeros = jnp.zeros((_NPAD, _D), jnp.float32)

    npad_e = _EPAD - _E
    pad_ix = jnp.arange(npad_e, dtype=jnp.int32)
    src_p = jnp.concatenate(
        [src, pad_ix % _N]).reshape(_NW * _BPW, _EBLK)
    dst_p = jnp.concatenate(
        [dst, _N + pad_ix % (_NPAD - _N)]).reshape(_NW * _BPW, _EBLK)

    hp1, dinv = _tc_first(x, deg_p, gb, W1)
    acc1 = _scatter_call(hp1, src_p, dst_p, zeros)
    hp2 = _tc_mid(acc1, hp1, dinv, b1.reshape(1, _D), W2)
    acc2 = _scatter_call(hp2, src_p, dst_p, zeros)
    hp3 = _tc_mid(acc2, hp2, dinv, b2.reshape(1, _D), W3)
    acc3 = _scatter_call(hp3, src_p, dst_p, zeros)
    return _tc_final(acc3, hp3, dinv, b3.reshape(1, _D), Wl,
                     bl.reshape(1, 2))


# speedup vs baseline: 3.5476x; 1.2530x over previous
import functools

import jax
import jax.numpy as jnp
from jax import lax
from jax.experimental import pallas as pl
from jax.experimental.pallas import tpu as pltpu
from jax.experimental.pallas import tpu_sc as plsc

_N = 10000
_E = 320000
_D = 128
_NC = 2
_NS = 16
_NW = _NC * _NS
_EPW = _E // _NW
_BLK = 128
_NBLK = _E // _BLK
_BASE_BLKS = _NBLK // _NW
_EXTRA_WORKERS = _NBLK % _NW
_NPAD = 10240
_RPS = _NPAD // _NS

_mesh = plsc.VectorSubcoreMesh(
    core_axis_name="c", subcore_axis_name="s", num_cores=_NC, num_subcores=_NS
)


def _deg_body(dst_hbm, out_hbm, idx_v, loc_v):
    c = lax.axis_index("c")
    s = lax.axis_index("s")
    wid = c * _NS + s

    def zero_body(i, carry):
        loc_v[pl.ds(i * 16, 16)] = jnp.zeros((16,), jnp.float32)
        return carry

    lax.fori_loop(0, _N // 16, zero_body, 0)

    pltpu.sync_copy(dst_hbm.at[pl.ds(wid * _EPW, _EPW)], idx_v)
    ones = jnp.ones((16,), jnp.float32)

    def hist_body(i, carry):
        idx = idx_v[pl.ds(i * 16, 16)]
        plsc.addupdate_scatter(loc_v, [idx], ones)
        return carry

    lax.fori_loop(0, _EPW // 16, hist_body, 0)
    pltpu.sync_copy(loc_v, out_hbm.at[wid])


_deg_call = pl.kernel(
    _deg_body,
    out_type=jax.ShapeDtypeStruct((_NW, _N), jnp.float32),
    mesh=_mesh,
    compiler_params=pltpu.CompilerParams(needs_layout_passes=False),
    scratch_types=[
        pltpu.VMEM((_EPW,), jnp.int32),
        pltpu.VMEM((_N,), jnp.float32),
    ],
)


_EBLK = 128
_BPW = 80
_EPAD = _BPW * _NW * _EBLK
_NBUF = 4


def _scatter_body(hp_hbm, src_hbm, dst_hbm, zeros_hbm, out_hbm,
                  srcix_v, dstix_v, r0, r1, r2, r3,
                  g0, g1, g2, g3, s0, s1, s2, s3, acc_sh):
    rows = (r0, r1, r2, r3)
    gsems = (g0, g1, g2, g3)
    ssems = (s0, s1, s2, s3)
    c = lax.axis_index("c")
    s = lax.axis_index("s")
    wid = c * _NS + s

    pltpu.sync_copy(src_hbm.at[pl.ds(wid * _BPW, _BPW)], srcix_v)
    pltpu.sync_copy(dst_hbm.at[pl.ds(wid * _BPW, _BPW)], dstix_v)

    pltpu.sync_copy(zeros_hbm.at[pl.ds(s * _RPS, _RPS)],
                    acc_sh.at[pl.ds(s * _RPS, _RPS)])
    plsc.subcore_barrier()

    for b in range(_NBUF):
        pltpu.async_copy(hp_hbm.at[srcix_v.at[b]], rows[b], gsems[b])

    nsteps = _BPW // _NBUF

    def step(si, carry):
        for b in range(_NBUF):
            j = si * _NBUF + b
            pltpu.make_async_copy(hp_hbm.at[srcix_v.at[j]],
                                  rows[b], gsems[b]).wait()
            sc = pltpu.async_copy(rows[b], acc_sh.at[dstix_v.at[j]],
                                  ssems[b], add=True)

            @pl.when(si < nsteps - 1)
            def _():
                sc.wait()
                pltpu.async_copy(hp_hbm.at[srcix_v.at[j + _NBUF]],
                                 rows[b], gsems[b])

        return carry

    lax.fori_loop(0, nsteps, step, 0)
    for b in range(_NBUF):
        pltpu.make_async_copy(rows[b], acc_sh.at[dstix_v.at[0]],
                              ssems[b]).wait()
    plsc.subcore_barrier()
    pltpu.sync_copy(acc_sh.at[pl.ds(s * _RPS, _RPS)],
                    out_hbm.at[c, pl.ds(s * _RPS, _RPS)])


_scatter_call = pl.kernel(
    _scatter_body,
    out_type=jax.ShapeDtypeStruct((_NC, _NPAD, _D), jnp.bfloat16),
    mesh=_mesh,
    compiler_params=pltpu.CompilerParams(
        needs_layout_passes=False, use_tc_tiling_on_sc=False),
    scratch_types=[
        pltpu.VMEM((_BPW, _EBLK), jnp.int32),
        pltpu.VMEM((_BPW, _EBLK), jnp.int32),
        pltpu.VMEM((_EBLK, _D), jnp.bfloat16),
        pltpu.VMEM((_EBLK, _D), jnp.bfloat16),
        pltpu.VMEM((_EBLK, _D), jnp.bfloat16),
        pltpu.VMEM((_EBLK, _D), jnp.bfloat16),
        pltpu.SemaphoreType.DMA,
        pltpu.SemaphoreType.DMA,
        pltpu.SemaphoreType.DMA,
        pltpu.SemaphoreType.DMA,
        pltpu.SemaphoreType.DMA,
        pltpu.SemaphoreType.DMA,
        pltpu.SemaphoreType.DMA,
        pltpu.SemaphoreType.DMA,
        pltpu.VMEM_SHARED((_NPAD, _D), jnp.bfloat16),
    ],
)


_RB = 1000
_GRID = _N // _RB


def _tc_first_body(x_ref, deg_ref, gb_ref, w_ref, hp_ref, dinv_ref):
    deg = jnp.sum(deg_ref[...], axis=0) + 1.0
    dinv = lax.rsqrt(deg)
    h = x_ref[...] * gb_ref[0:1, :] + gb_ref[1:2, :]
    hp = jnp.dot(h, w_ref[...], preferred_element_type=jnp.float32)
    hp_ref[...] = (hp * dinv).astype(jnp.bfloat16)
    dinv_ref[...] = dinv


def _tc_first(x, deg_p, gb, w1):
    return pl.pallas_call(
        _tc_first_body,
        grid=(_GRID,),
        in_specs=[
            pl.BlockSpec((_RB, _D), lambda i: (i, 0)),
            pl.BlockSpec((_NW, _RB, 1), lambda i: (0, i, 0)),
            pl.BlockSpec((2, _D), lambda i: (0, 0)),
            pl.BlockSpec((_D, _D), lambda i: (0, 0)),
        ],
        out_specs=[
            pl.BlockSpec((_RB, _D), lambda i: (i, 0)),
            pl.BlockSpec((_RB, 1), lambda i: (i, 0)),
        ],
        out_shape=[
            jax.ShapeDtypeStruct((_N, _D), jnp.bfloat16),
            jax.ShapeDtypeStruct((_N, 1), jnp.float32),
        ],
    )(x, deg_p, gb, w1)


def _tc_mid_body(acc_ref, hp_ref, dinv_ref, b_ref, w_ref, out_ref):
    dinv = dinv_ref[...]
    tot = (acc_ref[0].astype(jnp.float32) + acc_ref[1].astype(jnp.float32)
           + hp_ref[...].astype(jnp.float32))
    z = tot * dinv + b_ref[...]
    h = jnp.maximum(z, 0.0)
    hp = jnp.dot(h, w_ref[...], preferred_element_type=jnp.float32) * dinv
    out_ref[...] = hp.astype(jnp.bfloat16)


def _tc_mid(acc_p, hp, dinv, b, w_next):
    return pl.pallas_call(
        _tc_mid_body,
        grid=(_GRID,),
        in_specs=[
            pl.BlockSpec((_NC, _RB, _D), lambda i: (0, i, 0)),
            pl.BlockSpec((_RB, _D), lambda i: (i, 0)),
            pl.BlockSpec((_RB, 1), lambda i: (i, 0)),
            pl.BlockSpec((1, _D), lambda i: (0, 0)),
            pl.BlockSpec((_D, _D), lambda i: (0, 0)),
        ],
        out_specs=pl.BlockSpec((_RB, _D), lambda i: (i, 0)),
        out_shape=jax.ShapeDtypeStruct((_N, _D), jnp.bfloat16),
    )(acc_p, hp, dinv, b, w_next)


def _tc_final_body(acc_ref, hp_ref, dinv_ref, b_ref, wl_ref, bl_ref,
                   out_ref, sum_ref):
    i = pl.program_id(0)
    tot = (acc_ref[0].astype(jnp.float32) + acc_ref[1].astype(jnp.float32)
           + hp_ref[...].astype(jnp.float32))
    z = tot * dinv_ref[...] + b_ref[...]
    h = jnp.maximum(z, 0.0)
    part = jnp.sum(h, axis=0, keepdims=True)

    @pl.when(i == 0)
    def _():
        sum_ref[...] = part

    @pl.when(i > 0)
    def _():
        sum_ref[...] += part

    @pl.when(i == _GRID - 1)
    def _():
        m = sum_ref[...] * (1.0 / _N)
        logits = jnp.dot(m, wl_ref[...], preferred_element_type=jnp.float32)
        logits = logits + bl_ref[...]
        zmax = jnp.max(logits, axis=1, keepdims=True)
        e = jnp.exp(logits - zmax)
        out_ref[...] = e / jnp.sum(e, axis=1, keepdims=True)


def _tc_final(acc_p, hp, dinv, b, wl, bl):
    return pl.pallas_call(
        _tc_final_body,
        grid=(_GRID,),
        in_specs=[
            pl.BlockSpec((_NC, _RB, _D), lambda i: (0, i, 0)),
            pl.BlockSpec((_RB, _D), lambda i: (i, 0)),
            pl.BlockSpec((_RB, 1), lambda i: (i, 0)),
            pl.BlockSpec((1, _D), lambda i: (0, 0)),
            pl.BlockSpec((_D, 2), lambda i: (0, 0)),
            pl.BlockSpec((1, 2), lambda i: (0, 0)),
        ],
        out_specs=pl.BlockSpec((1, 2), lambda i: (0, 0)),
        out_shape=jax.ShapeDtypeStruct((1, 2), jnp.float32),
        scratch_shapes=[pltpu.VMEM((1, _D), jnp.float32)],
    )(acc_p, hp, dinv, b, wl, bl)


def kernel(x, edge_index, bn_gamma, bn_beta, W1, b1, W2, b2, W3, b3, Wl, bl):
    ei = edge_index.astype(jnp.int32)
    src = ei[0]
    dst = ei[1]

    eps = 1e-5
    gb = jnp.stack([bn_gamma * (1.0 / jnp.sqrt(1.0 + eps)), bn_beta])

    deg_p = _deg_call(dst).reshape(_NW, _N, 1)
    zeros = jnp.zeros((_NPAD, _D), jnp.bfloat16)

    npad_e = _EPAD - _E
    pad_ix = jnp.arange(npad_e, dtype=jnp.int32)
    src_p = jnp.concatenate(
        [src, pad_ix % _N]).reshape(_NW * _BPW, _EBLK)
    dst_p = jnp.concatenate(
        [dst, _N + pad_ix % (_NPAD - _N)]).reshape(_NW * _BPW, _EBLK)

    hp1, dinv = _tc_first(x, deg_p, gb, W1)
    acc1 = _scatter_call(hp1, src_p, dst_p, zeros)
    hp2 = _tc_mid(acc1, hp1, dinv, b1.reshape(1, _D), W2)
    acc2 = _scatter_call(hp2, src_p, dst_p, zeros)
    hp3 = _tc_mid(acc2, hp2, dinv, b2.reshape(1, _D), W3)
    acc3 = _scatter_call(hp3, src_p, dst_p, zeros)
    return _tc_final(acc3, hp3, dinv, b3.reshape(1, _D), Wl,
                     bl.reshape(1, 2))


# speedup vs baseline: 4.9485x; 1.3949x over previous
import jax
import jax.numpy as jnp
import numpy as np
from jax import lax
from jax.experimental import pallas as pl
from jax.experimental.pallas import tpu as pltpu
from jax.experimental.pallas import tpu_sc as plsc

_N = 10000
_E = 320000
_D = 128
_NC = 2
_NS = 16
_NW = _NC * _NS
_EBLK = 128
_BPW = 80
_EPAD = _BPW * _NW * _EBLK
_EPW = _EPAD // _NW
_NPAD = 10240
_RPS = _NPAD // _NS
_NBUF = 4

_PAD_EDGES = np.stack([
    (np.arange(_EPAD - _E) * 7) % _N,
    _N + np.arange(_EPAD - _E) % (_NPAD - _N),
]).astype(np.int32)

_mesh = plsc.VectorSubcoreMesh(
    core_axis_name="c", subcore_axis_name="s", num_cores=_NC, num_subcores=_NS
)


def _deg_body(ei_hbm, out_hbm, idx_v, loc_v):
    c = lax.axis_index("c")
    s = lax.axis_index("s")
    wid = c * _NS + s

    def zero_body(i, carry):
        loc_v[pl.ds(i * 16, 16)] = jnp.zeros((16,), jnp.float32)
        return carry

    lax.fori_loop(0, _NPAD // 16, zero_body, 0)

    pltpu.sync_copy(ei_hbm.at[1, pl.ds(wid * _BPW, _BPW)], idx_v)
    ones = jnp.ones((16,), jnp.float32)

    def hist_body(i, carry):
        r = i // 8
        k = i % 8
        idx = idx_v[r, pl.ds(k * 16, 16)]
        plsc.addupdate_scatter(loc_v, [idx], ones)
        return carry

    lax.fori_loop(0, _EPW // 16, hist_body, 0)
    pltpu.sync_copy(loc_v, out_hbm.at[wid])


_deg_call = pl.kernel(
    _deg_body,
    out_type=jax.ShapeDtypeStruct((_NW, _NPAD), jnp.float32),
    mesh=_mesh,
    compiler_params=pltpu.CompilerParams(needs_layout_passes=False),
    scratch_types=[
        pltpu.VMEM((_BPW, _EBLK), jnp.int32),
        pltpu.VMEM((_NPAD,), jnp.float32),
    ],
)


def _scatter_body(hp_hbm, ei_hbm, zeros_hbm, out_hbm,
                  srcix_v, dstix_v, r0, r1, r2, r3,
                  g0, g1, g2, g3, s0, s1, s2, s3, acc_sh):
    rows = (r0, r1, r2, r3)
    gsems = (g0, g1, g2, g3)
    ssems = (s0, s1, s2, s3)
    c = lax.axis_index("c")
    s = lax.axis_index("s")
    wid = c * _NS + s

    pltpu.sync_copy(ei_hbm.at[0, pl.ds(wid * _BPW, _BPW)], srcix_v)
    pltpu.sync_copy(ei_hbm.at[1, pl.ds(wid * _BPW, _BPW)], dstix_v)

    pltpu.sync_copy(zeros_hbm.at[pl.ds(s * _RPS, _RPS)],
                    acc_sh.at[pl.ds(s * _RPS, _RPS)])
    plsc.subcore_barrier()

    for b in range(_NBUF):
        pltpu.async_copy(hp_hbm.at[srcix_v.at[b]], rows[b], gsems[b])

    nsteps = _BPW // _NBUF

    def step(si, carry):
        for b in range(_NBUF):
            j = si * _NBUF + b
            pltpu.make_async_copy(hp_hbm.at[srcix_v.at[j]],
                                  rows[b], gsems[b]).wait()
            sc = pltpu.async_copy(rows[b], acc_sh.at[dstix_v.at[j]],
                                  ssems[b], add=True)

            @pl.when(si < nsteps - 1)
            def _():
                sc.wait()
                pltpu.async_copy(hp_hbm.at[srcix_v.at[j + _NBUF]],
                                 rows[b], gsems[b])

        return carry

    lax.fori_loop(0, nsteps, step, 0)
    for b in range(_NBUF):
        pltpu.make_async_copy(rows[b], acc_sh.at[dstix_v.at[0]],
                              ssems[b]).wait()
    plsc.subcore_barrier()
    pltpu.sync_copy(acc_sh.at[pl.ds(s * _RPS, _RPS)],
                    out_hbm.at[c, pl.ds(s * _RPS, _RPS)])


_scatter_call = pl.kernel(
    _scatter_body,
    out_type=jax.ShapeDtypeStruct((_NC, _NPAD, _D), jnp.bfloat16),
    mesh=_mesh,
    compiler_params=pltpu.CompilerParams(
        needs_layout_passes=False, use_tc_tiling_on_sc=False),
    scratch_types=[
        pltpu.VMEM((_BPW, _EBLK), jnp.int32),
        pltpu.VMEM((_BPW, _EBLK), jnp.int32),
        pltpu.VMEM((_EBLK, _D), jnp.bfloat16),
        pltpu.VMEM((_EBLK, _D), jnp.bfloat16),
        pltpu.VMEM((_EBLK, _D), jnp.bfloat16),
        pltpu.VMEM((_EBLK, _D), jnp.bfloat16),
        pltpu.SemaphoreType.DMA,
        pltpu.SemaphoreType.DMA,
        pltpu.SemaphoreType.DMA,
        pltpu.SemaphoreType.DMA,
        pltpu.SemaphoreType.DMA,
        pltpu.SemaphoreType.DMA,
        pltpu.SemaphoreType.DMA,
        pltpu.SemaphoreType.DMA,
        pltpu.VMEM_SHARED((_NPAD, _D), jnp.bfloat16),
    ],
)


_RB = 1024
_GRID = 10


def _dinv_col(deg_blk):
    dcol = lax.dot_general(deg_blk, jnp.ones((_NW, 1), jnp.float32),
                           (((0,), (0,)), ((), ())),
                           preferred_element_type=jnp.float32)
    return lax.rsqrt(dcol + 1.0)


def _tc_first_body(x_ref, deg_ref, gb_ref, w_ref, hp_ref, dinvb_ref):
    dinv = _dinv_col(deg_ref[...])
    h = x_ref[...] * gb_ref[0:1, :] + gb_ref[1:2, :]
    hp = jnp.dot(h, w_ref[...], preferred_element_type=jnp.float32)
    hp_ref[...] = (hp * dinv).astype(jnp.bfloat16)
    dinvb_ref[...] = (dinv * jnp.ones((1, _D), jnp.float32)).astype(
        jnp.bfloat16)


def _tc_first(x, deg_p, gb, w1):
    return pl.pallas_call(
        _tc_first_body,
        grid=(_GRID,),
        in_specs=[
            pl.BlockSpec((_RB, _D), lambda i: (i, 0)),
            pl.BlockSpec((_NW, _RB), lambda i: (0, i)),
            pl.BlockSpec((2, _D), lambda i: (0, 0)),
            pl.BlockSpec((_D, _D), lambda i: (0, 0)),
        ],
        out_specs=[
            pl.BlockSpec((_RB, _D), lambda i: (i, 0)),
            pl.BlockSpec((_RB, _D), lambda i: (i, 0)),
        ],
        out_shape=[
            jax.ShapeDtypeStruct((_N, _D), jnp.bfloat16),
            jax.ShapeDtypeStruct((_N, _D), jnp.bfloat16),
        ],
    )(x, deg_p, gb, w1)


def _tc_mid_body(acc_ref, hp_ref, dinvb_ref, b_ref, w_ref, out_ref):
    dinv = dinvb_ref[...].astype(jnp.float32)
    tot = (acc_ref[0].astype(jnp.float32) + acc_ref[1].astype(jnp.float32)
           + hp_ref[...].astype(jnp.float32))
    z = tot * dinv + b_ref[...]
    h = jnp.maximum(z, 0.0)
    hp = jnp.dot(h, w_ref[...], preferred_element_type=jnp.float32) * dinv
    out_ref[...] = hp.astype(jnp.bfloat16)


def _tc_mid(acc_p, hp, dinvb, b, w_next):
    return pl.pallas_call(
        _tc_mid_body,
        grid=(_GRID,),
        in_specs=[
            pl.BlockSpec((_NC, _RB, _D), lambda i: (0, i, 0)),
            pl.BlockSpec((_RB, _D), lambda i: (i, 0)),
            pl.BlockSpec((_RB, _D), lambda i: (i, 0)),
            pl.BlockSpec((1, _D), lambda i: (0, 0)),
            pl.BlockSpec((_D, _D), lambda i: (0, 0)),
        ],
        out_specs=pl.BlockSpec((_RB, _D), lambda i: (i, 0)),
        out_shape=jax.ShapeDtypeStruct((_N, _D), jnp.bfloat16),
    )(acc_p, hp, dinvb, b, w_next)


def _tc_final_body(acc_ref, hp_ref, dinvb_ref, b_ref, wl_ref, bl_ref,
                   out_ref, sum_ref):
    i = pl.program_id(0)
    dinv = dinvb_ref[...].astype(jnp.float32)
    tot = (acc_ref[0].astype(jnp.float32) + acc_ref[1].astype(jnp.float32)
           + hp_ref[...].astype(jnp.float32))
    z = tot * dinv + b_ref[...]
    h = jnp.maximum(z, 0.0)
    rowid = lax.broadcasted_iota(jnp.int32, (_RB, _D), 0)
    h = jnp.where(rowid < _N - i * _RB, h, 0.0)
    part = jnp.sum(h, axis=0, keepdims=True)

    @pl.when(i == 0)
    def _():
        sum_ref[...] = part

    @pl.when(i > 0)
    def _():
        sum_ref[...] += part

    @pl.when(i == _GRID - 1)
    def _():
        m = sum_ref[...] * (1.0 / _N)
        logits = jnp.dot(m, wl_ref[...], preferred_element_type=jnp.float32)
        logits = logits + bl_ref[...]
        zmax = jnp.max(logits, axis=1, keepdims=True)
        e = jnp.exp(logits - zmax)
        out_ref[...] = e / jnp.sum(e, axis=1, keepdims=True)


def _tc_final(acc_p, hp, dinvb, b, wl, bl):
    return pl.pallas_call(
        _tc_final_body,
        grid=(_GRID,),
        in_specs=[
            pl.BlockSpec((_NC, _RB, _D), lambda i: (0, i, 0)),
            pl.BlockSpec((_RB, _D), lambda i: (i, 0)),
            pl.BlockSpec((_RB, _D), lambda i: (i, 0)),
            pl.BlockSpec((1, _D), lambda i: (0, 0)),
            pl.BlockSpec((_D, 2), lambda i: (0, 0)),
            pl.BlockSpec((1, 2), lambda i: (0, 0)),
        ],
        out_specs=pl.BlockSpec((1, 2), lambda i: (0, 0)),
        out_shape=jax.ShapeDtypeStruct((1, 2), jnp.float32),
        scratch_shapes=[pltpu.VMEM((1, _D), jnp.float32)],
    )(acc_p, hp, dinvb, b, wl, bl)


def kernel(x, edge_index, bn_gamma, bn_beta, W1, b1, W2, b2, W3, b3, Wl, bl):
    ei = edge_index.astype(jnp.int32)
    ei_p = jnp.concatenate(
        [ei, jnp.asarray(_PAD_EDGES)], axis=1).reshape(2, _NW * _BPW, _EBLK)

    eps = 1e-5
    gb = jnp.stack([bn_gamma * (1.0 / jnp.sqrt(1.0 + eps)), bn_beta])

    deg_p = _deg_call(ei_p)
    zeros = jnp.zeros((_NPAD, _D), jnp.bfloat16)

    hp1, dinvb = _tc_first(x, deg_p, gb, W1)
    acc1 = _scatter_call(hp1, ei_p, zeros)
    hp2 = _tc_mid(acc1, hp1, dinvb, b1.reshape(1, _D), W2)
    acc2 = _scatter_call(hp2, ei_p, zeros)
    hp3 = _tc_mid(acc2, hp2, dinvb, b2.reshape(1, _D), W3)
    acc3 = _scatter_call(hp3, ei_p, zeros)
    return _tc_final(acc3, hp3, dinvb, b3.reshape(1, _D), Wl,
                     bl.reshape(1, 2))


# speedup vs baseline: 5.0086x; 1.0121x over previous
import jax
import jax.numpy as jnp
import numpy as np
from jax import lax
from jax.experimental import pallas as pl
from jax.experimental.pallas import tpu as pltpu
from jax.experimental.pallas import tpu_sc as plsc

_N = 10000
_E = 320000
_D = 128
_NC = 2
_NS = 16
_NW = _NC * _NS
_EBLK = 128
_BPW = 80
_EPAD = _BPW * _NW * _EBLK
_EPW = _EPAD // _NW
_NPAD = 10240
_RPS = _NPAD // _NS
_NBUF = 5

_PAD_EDGES = np.stack([
    (np.arange(_EPAD - _E) * 7) % _N,
    _N + np.arange(_EPAD - _E) % (_NPAD - _N),
]).astype(np.int32)

_mesh = plsc.VectorSubcoreMesh(
    core_axis_name="c", subcore_axis_name="s", num_cores=_NC, num_subcores=_NS
)


def _deg_body(ei_hbm, out_hbm, idx_v, loc_v):
    c = lax.axis_index("c")
    s = lax.axis_index("s")
    wid = c * _NS + s

    def zero_body(i, carry):
        loc_v[pl.ds(i * 16, 16)] = jnp.zeros((16,), jnp.float32)
        return carry

    lax.fori_loop(0, _NPAD // 16, zero_body, 0)

    pltpu.sync_copy(ei_hbm.at[1, pl.ds(wid * _BPW, _BPW)], idx_v)
    ones = jnp.ones((16,), jnp.float32)

    def hist_body(i, carry):
        r = i // 8
        k = i % 8
        idx = idx_v[r, pl.ds(k * 16, 16)]
        plsc.addupdate_scatter(loc_v, [idx], ones)
        return carry

    lax.fori_loop(0, _EPW // 16, hist_body, 0)
    pltpu.sync_copy(loc_v, out_hbm.at[wid])


_deg_call = pl.kernel(
    _deg_body,
    out_type=jax.ShapeDtypeStruct((_NW, _NPAD), jnp.float32),
    mesh=_mesh,
    compiler_params=pltpu.CompilerParams(needs_layout_passes=False),
    scratch_types=[
        pltpu.VMEM((_BPW, _EBLK), jnp.int32),
        pltpu.VMEM((_NPAD,), jnp.float32),
    ],
)


def _scatter_body(hp_hbm, ei_hbm, out_hbm,
                  srcix_v, dstix_v, zbuf_v, r0, r1, r2, r3, r4,
                  g0, g1, g2, g3, g4, s0, s1, s2, s3, s4, acc_sh):
    rows = (r0, r1, r2, r3, r4)
    gsems = (g0, g1, g2, g3, g4)
    ssems = (s0, s1, s2, s3, s4)
    c = lax.axis_index("c")
    s = lax.axis_index("s")
    wid = c * _NS + s

    pltpu.sync_copy(ei_hbm.at[0, pl.ds(wid * _BPW, _BPW)], srcix_v)
    pltpu.sync_copy(ei_hbm.at[1, pl.ds(wid * _BPW, _BPW)], dstix_v)

    zero32 = jnp.zeros((32,), jnp.bfloat16)

    def zrow(r, carry):
        for k in range(4):
            zbuf_v[r, pl.ds(k * 32, 32)] = zero32
        return carry

    lax.fori_loop(0, _EBLK, zrow, 0)
    for k in range(_RPS // _EBLK):
        pltpu.sync_copy(zbuf_v, acc_sh.at[pl.ds(s * _RPS + k * _EBLK, _EBLK)])
    plsc.subcore_barrier()

    for b in range(_NBUF):
        pltpu.async_copy(hp_hbm.at[srcix_v.at[b]], rows[b], gsems[b])

    nsteps = _BPW // _NBUF

    def step(si, carry):
        for b in range(_NBUF):
            j = si * _NBUF + b
            pltpu.make_async_copy(hp_hbm.at[srcix_v.at[j]],
                                  rows[b], gsems[b]).wait()
            sc = pltpu.async_copy(rows[b], acc_sh.at[dstix_v.at[j]],
                                  ssems[b], add=True)

            @pl.when(si < nsteps - 1)
            def _():
                sc.wait()
                pltpu.async_copy(hp_hbm.at[srcix_v.at[j + _NBUF]],
                                 rows[b], gsems[b])

        return carry

    lax.fori_loop(0, nsteps, step, 0)
    for b in range(_NBUF):
        pltpu.make_async_copy(rows[b], acc_sh.at[dstix_v.at[0]],
                              ssems[b]).wait()
    plsc.subcore_barrier()
    pltpu.sync_copy(acc_sh.at[pl.ds(s * _RPS, _RPS)],
                    out_hbm.at[c, pl.ds(s * _RPS, _RPS)])


_scatter_call = pl.kernel(
    _scatter_body,
    out_type=jax.ShapeDtypeStruct((_NC, _NPAD, _D), jnp.bfloat16),
    mesh=_mesh,
    compiler_params=pltpu.CompilerParams(
        needs_layout_passes=False, use_tc_tiling_on_sc=False),
    scratch_types=[
        pltpu.VMEM((_BPW, _EBLK), jnp.int32),
        pltpu.VMEM((_BPW, _EBLK), jnp.int32),
        pltpu.VMEM((_EBLK, _D), jnp.bfloat16),
        pltpu.VMEM((_EBLK, _D), jnp.bfloat16),
        pltpu.VMEM((_EBLK, _D), jnp.bfloat16),
        pltpu.VMEM((_EBLK, _D), jnp.bfloat16),
        pltpu.VMEM((_EBLK, _D), jnp.bfloat16),
        pltpu.VMEM((_EBLK, _D), jnp.bfloat16),
        pltpu.SemaphoreType.DMA,
        pltpu.SemaphoreType.DMA,
        pltpu.SemaphoreType.DMA,
        pltpu.SemaphoreType.DMA,
        pltpu.SemaphoreType.DMA,
        pltpu.SemaphoreType.DMA,
        pltpu.SemaphoreType.DMA,
        pltpu.SemaphoreType.DMA,
        pltpu.SemaphoreType.DMA,
        pltpu.SemaphoreType.DMA,
        pltpu.VMEM_SHARED((_NPAD, _D), jnp.bfloat16),
    ],
)


_RB = 1024
_GRID = 10


def _dinv_col(deg_blk):
    dcol = lax.dot_general(deg_blk, jnp.ones((_NW, 1), jnp.float32),
                           (((0,), (0,)), ((), ())),
                           preferred_element_type=jnp.float32)
    return lax.rsqrt(dcol + 1.0)


def _tc_first_body(x_ref, deg_ref, gb_ref, w_ref, hp_ref, dinvb_ref):
    dinv = _dinv_col(deg_ref[...])
    h = x_ref[...] * gb_ref[0:1, :] + gb_ref[1:2, :]
    hp = jnp.dot(h, w_ref[...], preferred_element_type=jnp.float32)
    hp_ref[...] = (hp * dinv).astype(jnp.bfloat16)
    dinvb_ref[...] = (dinv * jnp.ones((1, _D), jnp.float32)).astype(
        jnp.bfloat16)


def _tc_first(x, deg_p, gb, w1):
    return pl.pallas_call(
        _tc_first_body,
        grid=(_GRID,),
        in_specs=[
            pl.BlockSpec((_RB, _D), lambda i: (i, 0)),
            pl.BlockSpec((_NW, _RB), lambda i: (0, i)),
            pl.BlockSpec((2, _D), lambda i: (0, 0)),
            pl.BlockSpec((_D, _D), lambda i: (0, 0)),
        ],
        out_specs=[
            pl.BlockSpec((_RB, _D), lambda i: (i, 0)),
            pl.BlockSpec((_RB, _D), lambda i: (i, 0)),
        ],
        out_shape=[
            jax.ShapeDtypeStruct((_N, _D), jnp.bfloat16),
            jax.ShapeDtypeStruct((_N, _D), jnp.bfloat16),
        ],
    )(x, deg_p, gb, w1)


def _tc_mid_body(acc_ref, hp_ref, dinvb_ref, b_ref, w_ref, out_ref):
    dinv = dinvb_ref[...].astype(jnp.float32)
    tot = (acc_ref[0].astype(jnp.float32) + acc_ref[1].astype(jnp.float32)
           + hp_ref[...].astype(jnp.float32))
    z = tot * dinv + b_ref[...]
    h = jnp.maximum(z, 0.0)
    hp = jnp.dot(h, w_ref[...], preferred_element_type=jnp.float32) * dinv
    out_ref[...] = hp.astype(jnp.bfloat16)


def _tc_mid(acc_p, hp, dinvb, b, w_next):
    return pl.pallas_call(
        _tc_mid_body,
        grid=(_GRID,),
        in_specs=[
            pl.BlockSpec((_NC, _RB, _D), lambda i: (0, i, 0)),
            pl.BlockSpec((_RB, _D), lambda i: (i, 0)),
            pl.BlockSpec((_RB, _D), lambda i: (i, 0)),
            pl.BlockSpec((1, _D), lambda i: (0, 0)),
            pl.BlockSpec((_D, _D), lambda i: (0, 0)),
        ],
        out_specs=pl.BlockSpec((_RB, _D), lambda i: (i, 0)),
        out_shape=jax.ShapeDtypeStruct((_N, _D), jnp.bfloat16),
    )(acc_p, hp, dinvb, b, w_next)


def _tc_final_body(acc_ref, hp_ref, dinvb_ref, b_ref, wl_ref, bl_ref,
                   out_ref, sum_ref):
    i = pl.program_id(0)
    dinv = dinvb_ref[...].astype(jnp.float32)
    tot = (acc_ref[0].astype(jnp.float32) + acc_ref[1].astype(jnp.float32)
           + hp_ref[...].astype(jnp.float32))
    z = tot * dinv + b_ref[...]
    h = jnp.maximum(z, 0.0)
    rowid = lax.broadcasted_iota(jnp.int32, (_RB, _D), 0)
    h = jnp.where(rowid < _N - i * _RB, h, 0.0)
    part = jnp.sum(h, axis=0, keepdims=True)

    @pl.when(i == 0)
    def _():
        sum_ref[...] = part

    @pl.when(i > 0)
    def _():
        sum_ref[...] += part

    @pl.when(i == _GRID - 1)
    def _():
        m = sum_ref[...] * (1.0 / _N)
        logits = jnp.dot(m, wl_ref[...], preferred_element_type=jnp.float32)
        logits = logits + bl_ref[...]
        zmax = jnp.max(logits, axis=1, keepdims=True)
        e = jnp.exp(logits - zmax)
        out_ref[...] = e / jnp.sum(e, axis=1, keepdims=True)


def _tc_final(acc_p, hp, dinvb, b, wl, bl):
    return pl.pallas_call(
        _tc_final_body,
        grid=(_GRID,),
        in_specs=[
            pl.BlockSpec((_NC, _RB, _D), lambda i: (0, i, 0)),
            pl.BlockSpec((_RB, _D), lambda i: (i, 0)),
            pl.BlockSpec((_RB, _D), lambda i: (i, 0)),
            pl.BlockSpec((1, _D), lambda i: (0, 0)),
            pl.BlockSpec((_D, 2), lambda i: (0, 0)),
            pl.BlockSpec((1, 2), lambda i: (0, 0)),
        ],
        out_specs=pl.BlockSpec((1, 2), lambda i: (0, 0)),
        out_shape=jax.ShapeDtypeStruct((1, 2), jnp.float32),
        scratch_shapes=[pltpu.VMEM((1, _D), jnp.float32)],
    )(acc_p, hp, dinvb, b, wl, bl)


def kernel(x, edge_index, bn_gamma, bn_beta, W1, b1, W2, b2, W3, b3, Wl, bl):
    ei = edge_index.astype(jnp.int32)
    ei_p = jnp.concatenate(
        [ei, jnp.asarray(_PAD_EDGES)], axis=1).reshape(2, _NW * _BPW, _EBLK)

    eps = 1e-5
    gb = jnp.stack([bn_gamma * (1.0 / jnp.sqrt(1.0 + eps)), bn_beta])

    deg_p = _deg_call(ei_p)

    hp1, dinvb = _tc_first(x, deg_p, gb, W1)
    acc1 = _scatter_call(hp1, ei_p)
    hp2 = _tc_mid(acc1, hp1, dinvb, b1.reshape(1, _D), W2)
    acc2 = _scatter_call(hp2, ei_p)
    hp3 = _tc_mid(acc2, hp2, dinvb, b2.reshape(1, _D), W3)
    acc3 = _scatter_call(hp3, ei_p)
    return _tc_final(acc3, hp3, dinvb, b3.reshape(1, _D), Wl,
                     bl.reshape(1, 2))
